# Initial kernel scaffold; baseline (speedup 1.0000x reference)
import jax
import jax.numpy as jnp
from jax.experimental import pallas as pl


def kernel(pid, pos_embeds):
    raise NotImplementedError("write your pallas kernel here")


# speedup vs baseline: 1.6197x; 1.6197x over previous
import functools

import jax
import jax.numpy as jnp
from jax import lax
from jax.experimental import pallas as pl
from jax.experimental.pallas import tpu as pltpu
from jax.experimental.pallas import tpu_sc as plsc

B, NPATCH, DIM = 64, 576, 768
L = 577
LANES = 16
CHUNK = 64
NCHUNK = NPATCH // CHUNK


def _sc_gather(pid_flat, table):
    info = plsc.get_sparse_core_info()
    nw = info.num_cores * info.num_subcores
    bpw = B // nw
    mesh = plsc.VectorSubcoreMesh(core_axis_name="c", subcore_axis_name="s")

    @functools.partial(
        pl.kernel,
        mesh=mesh,
        out_type=jax.ShapeDtypeStruct((B, L, DIM), jnp.float32),
        scratch_types=[
            pltpu.VMEM((NPATCH,), jnp.int32),
            pltpu.VMEM((NPATCH + LANES,), jnp.int32),
            pltpu.VMEM((CHUNK, DIM), jnp.float32),
            pltpu.VMEM((CHUNK, DIM), jnp.float32),
            pltpu.VMEM((1, DIM), jnp.float32),
            pltpu.SemaphoreType.DMA,
            pltpu.SemaphoreType.DMA,
        ],
    )
    def run(pid_hbm, table_hbm, out_hbm, pidv, idxv, rows0, rows1, lastv, sem0, sem1):
        wid = lax.axis_index("s") * info.num_cores + lax.axis_index("c")
        lane = lax.iota(jnp.int32, LANES)
        rows = (rows0, rows1)
        sems = (sem0, sem1)
        for j in range(bpw):
            b = wid * bpw + j
            base = pl.multiple_of(b * NPATCH, NPATCH)
            pltpu.sync_copy(pid_hbm.at[pl.ds(base, NPATCH)], pidv)
            prev_last = jnp.full((LANES,), -1, dtype=jnp.int32)
            perm = (lane + (LANES - 1)) & (LANES - 1)
            for k in range(NPATCH // LANES):
                cur = pidv[pl.ds(k * LANES, LANES)]
                cur_rot = cur.at[perm].get(mode="promise_in_bounds")
                shifted = jnp.where(lane == 0, prev_last, cur_rot) + 1
                idxv[pl.ds(k * LANES, LANES)] = shifted
                prev_last = cur_rot.at[perm * 0].get(mode="promise_in_bounds")
            idxv[pl.ds(NPATCH, LANES)] = prev_last + 1
            for c in range(NCHUNK):
                buf = rows[c % 2]
                pltpu.async_copy(
                    table_hbm.at[idxv.at[pl.ds(c * CHUNK, CHUNK)]], buf, sems[c % 2]
                ).wait()
                pltpu.sync_copy(buf, out_hbm.at[b, pl.ds(c * CHUNK, CHUNK)])
            pltpu.async_copy(
                table_hbm.at[idxv.at[pl.ds(NCHUNK * CHUNK, 1)]], lastv, sem0
            ).wait()
            pltpu.sync_copy(lastv, out_hbm.at[b, pl.ds(NCHUNK * CHUNK, 1)])

    return run(pid_flat, table)


def kernel(pid, pos_embeds):
    return _sc_gather(pid.astype(jnp.int32).reshape(-1), pos_embeds)


# speedup vs baseline: 1.7085x; 1.0548x over previous
import functools

import jax
import jax.numpy as jnp
from jax import lax
from jax.experimental import pallas as pl
from jax.experimental.pallas import tpu as pltpu
from jax.experimental.pallas import tpu_sc as plsc

B, NPATCH, DIM = 64, 576, 768
L = 577
LANES = 16
CHUNK = 64
NCHUNK = NPATCH // CHUNK
IDXSTRIDE = NPATCH + LANES


def _sc_gather(pid_flat, table):
    info = plsc.get_sparse_core_info()
    nw = info.num_cores * info.num_subcores
    bpw = B // nw
    mesh = plsc.VectorSubcoreMesh(core_axis_name="c", subcore_axis_name="s")

    @functools.partial(
        pl.kernel,
        mesh=mesh,
        out_type=jax.ShapeDtypeStruct((B, L, DIM), jnp.float32),
        scratch_types=[
            pltpu.VMEM((bpw * NPATCH,), jnp.int32),
            pltpu.VMEM((bpw * IDXSTRIDE,), jnp.int32),
            pltpu.VMEM((CHUNK, DIM), jnp.float32),
            pltpu.VMEM((CHUNK, DIM), jnp.float32),
            pltpu.VMEM((1, DIM), jnp.float32),
            pltpu.VMEM((1, DIM), jnp.float32),
            pltpu.SemaphoreType.DMA,
            pltpu.SemaphoreType.DMA,
            pltpu.SemaphoreType.DMA,
            pltpu.SemaphoreType.DMA,
            pltpu.SemaphoreType.DMA,
        ],
    )
    def run(pid_hbm, table_hbm, out_hbm, pidv, idxv,
            rows0, rows1, last0, last1, gs0, gs1, ws0, ws1, ls):
        wid = lax.axis_index("s") * info.num_cores + lax.axis_index("c")
        lane = lax.iota(jnp.int32, LANES)
        perm = (lane + (LANES - 1)) & (LANES - 1)
        rows, lasts = (rows0, rows1), (last0, last1)
        gsems, wsems = (gs0, gs1), (ws0, ws1)

        base = pl.multiple_of(wid * (bpw * NPATCH), bpw * NPATCH)
        pltpu.sync_copy(pid_hbm.at[pl.ds(base, bpw * NPATCH)], pidv)

        for j in range(bpw):
            prev_last = jnp.full((LANES,), -1, dtype=jnp.int32)
            for k in range(NPATCH // LANES):
                cur = pidv[pl.ds(j * NPATCH + k * LANES, LANES)]
                cur_rot = cur.at[perm].get(mode="promise_in_bounds")
                shifted = jnp.where(lane == 0, prev_last, cur_rot) + 1
                idxv[pl.ds(j * IDXSTRIDE + k * LANES, LANES)] = shifted
                prev_last = cur_rot.at[perm * 0].get(mode="promise_in_bounds")
            idxv[pl.ds(j * IDXSTRIDE + NPATCH, LANES)] = prev_last + 1

        chunks = [(j, c) for j in range(bpw) for c in range(NCHUNK)]
        nt = len(chunks)
        g = [None] * nt
        w = [None] * nt
        for t, (j, c) in enumerate(chunks):
            if t >= 2:
                w[t - 2].wait()
            g[t] = pltpu.async_copy(
                table_hbm.at[idxv.at[pl.ds(j * IDXSTRIDE + c * CHUNK, CHUNK)]],
                rows[t % 2],
                gsems[t % 2],
            )
            if t >= 1:
                jp, cp = chunks[t - 1]
                g[t - 1].wait()
                w[t - 1] = pltpu.async_copy(
                    rows[(t - 1) % 2],
                    out_hbm.at[wid * bpw + jp, pl.ds(cp * CHUNK, CHUNK)],
                    wsems[(t - 1) % 2],
                )
        jl, cl = chunks[nt - 1]
        g[nt - 1].wait()
        w[nt - 1] = pltpu.async_copy(
            rows[(nt - 1) % 2],
            out_hbm.at[wid * bpw + jl, pl.ds(cl * CHUNK, CHUNK)],
            wsems[(nt - 1) % 2],
        )
        fg = [
            pltpu.async_copy(
                table_hbm.at[idxv.at[pl.ds(j * IDXSTRIDE + NCHUNK * CHUNK, 1)]],
                lasts[j],
                ls,
            )
            for j in range(bpw)
        ]
        for j in range(bpw):
            fg[j].wait()
            pltpu.sync_copy(
                lasts[j], out_hbm.at[wid * bpw + j, pl.ds(NCHUNK * CHUNK, 1)]
            )
        w[nt - 2].wait()
        w[nt - 1].wait()

    return run(pid_flat, table)


def kernel(pid, pos_embeds):
    return _sc_gather(pid.astype(jnp.int32).reshape(-1), pos_embeds)


# speedup vs baseline: 2.9261x; 1.7127x over previous
import functools

import jax
import jax.numpy as jnp
from jax import lax
from jax.experimental import pallas as pl
from jax.experimental.pallas import tpu as pltpu
from jax.experimental.pallas import tpu_sc as plsc

B, NPATCH, DIM = 64, 576, 768
L = 577
LPW = 18


def _sc_gather(pcol_flat, table):
    info = plsc.get_sparse_core_info()
    nw = info.num_cores * info.num_subcores
    mesh = plsc.VectorSubcoreMesh(core_axis_name="c", subcore_axis_name="s")

    @functools.partial(
        pl.kernel,
        mesh=mesh,
        out_type=jax.ShapeDtypeStruct((L, B, DIM), jnp.float32),
        scratch_types=[
            pltpu.VMEM((LPW * B,), jnp.int32),
            pltpu.VMEM((B,), jnp.int32),
            pltpu.VMEM((B, DIM), jnp.float32),
            pltpu.VMEM((B, DIM), jnp.float32),
            pltpu.SemaphoreType.DMA,
            pltpu.SemaphoreType.DMA,
            pltpu.SemaphoreType.DMA,
            pltpu.SemaphoreType.DMA,
        ],
    )
    def run(pcol_hbm, table_hbm, out_hbm, idxv, tidxv, rows0, rows1,
            gs0, gs1, ws0, ws1):
        wid = lax.axis_index("s") * info.num_cores + lax.axis_index("c")
        l0 = wid * LPW
        rows, gsems, wsems = (rows0, rows1), (gs0, gs1), (ws0, ws1)

        base = pl.multiple_of(wid * (LPW * B), LPW * B)
        pltpu.sync_copy(pcol_hbm.at[pl.ds(base, LPW * B)], idxv)

        g = [None] * LPW
        w = [None] * LPW
        for t in range(LPW):
            if t >= 2:
                w[t - 2].wait()
            g[t] = pltpu.async_copy(
                table_hbm.at[idxv.at[pl.ds(t * B, B)]], rows[t % 2], gsems[t % 2]
            )
            if t >= 1:
                g[t - 1].wait()
                w[t - 1] = pltpu.async_copy(
                    rows[(t - 1) % 2], out_hbm.at[l0 + t - 1], wsems[(t - 1) % 2]
                )
        g[LPW - 1].wait()
        w[LPW - 1] = pltpu.async_copy(
            rows[(LPW - 1) % 2], out_hbm.at[l0 + LPW - 1], wsems[(LPW - 1) % 2]
        )
        w[LPW - 2].wait()
        @pl.when(wid == nw - 1)
        def _tail():
            pltpu.sync_copy(pcol_hbm.at[pl.ds(nw * LPW * B, B)], tidxv)
            pltpu.async_copy(table_hbm.at[tidxv], rows0, gs0).wait()
            pltpu.sync_copy(rows0, out_hbm.at[nw * LPW])
        w[LPW - 1].wait()

    return run(pcol_flat, table)


def kernel(pid, pos_embeds):
    p = jnp.pad(pid.astype(jnp.int32).T + 1, ((1, 0), (0, 0)))
    out_t = _sc_gather(p.reshape(-1), pos_embeds)
    return jnp.transpose(out_t, (1, 0, 2))
